# baseline (speedup 1.0000x reference)
import functools

import jax
import jax.numpy as jnp
from jax import lax
from jax.experimental import pallas as pl
from jax.experimental.pallas import tpu as pltpu
from jax.experimental.pallas import tpu_sc as plsc

K = 100000
DIM = 128
B = 16384

_INFO = plsc.get_sparse_core_info()
_NC, _NS = _INFO.num_cores, _INFO.num_subcores
_NW = _NC * _NS

_VAL_PER_W = B // _NW
_TAIL = K - B
_TAIL_PER_W = ((_TAIL + _NW - 1) // _NW + 7) // 8 * 8


@functools.partial(
    pl.kernel,
    mesh=plsc.VectorSubcoreMesh(core_axis_name="c", subcore_axis_name="s"),
    out_type=jax.ShapeDtypeStruct((K, DIM), jnp.float32),
    scratch_types=[pltpu.SemaphoreType.DMA, pltpu.SemaphoreType.DMA],
)
def _push(bank_hbm, value_hbm, out_hbm, sem_v, sem_t):
    wid = lax.axis_index("s") * _NC + lax.axis_index("c")
    vb = wid * _VAL_PER_W
    cp_v = pltpu.async_copy(
        value_hbm.at[pl.ds(vb, _VAL_PER_W)],
        out_hbm.at[pl.ds(vb, _VAL_PER_W)],
        sem_v,
    )
    tb = jnp.minimum(B + wid * _TAIL_PER_W, K - _TAIL_PER_W)
    tb = pl.multiple_of(tb, 8)
    cp_t = pltpu.async_copy(
        bank_hbm.at[pl.ds(tb, _TAIL_PER_W)],
        out_hbm.at[pl.ds(tb, _TAIL_PER_W)],
        sem_t,
    )
    cp_v.wait()
    cp_t.wait()


def kernel(bank, value):
    return _push(bank, value)


# speedup vs baseline: 1.0089x; 1.0089x over previous
import functools

import jax
import jax.numpy as jnp
from jax.experimental import pallas as pl
from jax.experimental.pallas import tpu as pltpu

K = 100000
DIM = 128
B = 16384

_VAL_CHUNKS = 2
_TAIL = K - B
_TAIL_CHUNKS = 12
_TAIL_CH = _TAIL // _TAIL_CHUNKS


def _push_body(bank_ref, value_ref, out_ref, *sems):
    copies = []
    vch = B // _VAL_CHUNKS
    for i in range(_VAL_CHUNKS):
        copies.append(pltpu.make_async_copy(
            value_ref.at[pl.ds(i * vch, vch)],
            out_ref.at[pl.ds(i * vch, vch)],
            sems[i],
        ))
    for i in range(_TAIL_CHUNKS):
        base = B + i * _TAIL_CH
        copies.append(pltpu.make_async_copy(
            bank_ref.at[pl.ds(base, _TAIL_CH)],
            out_ref.at[pl.ds(base, _TAIL_CH)],
            sems[_VAL_CHUNKS + i],
        ))
    for c in copies:
        c.start()
    for c in copies:
        c.wait()


@jax.jit
def kernel(bank, value):
    return pl.pallas_call(
        _push_body,
        out_shape=jax.ShapeDtypeStruct((K, DIM), jnp.float32),
        in_specs=[
            pl.BlockSpec(memory_space=pl.ANY),
            pl.BlockSpec(memory_space=pl.ANY),
        ],
        out_specs=pl.BlockSpec(memory_space=pl.ANY),
        scratch_shapes=[pltpu.SemaphoreType.DMA] * (_VAL_CHUNKS + _TAIL_CHUNKS),
    )(bank, value)


# speedup vs baseline: 21.7714x; 21.5786x over previous
import jax
import jax.numpy as jnp
from jax.experimental import pallas as pl
from jax.experimental.pallas import tpu as pltpu

K = 100000
DIM = 128
B = 16384

_BR = 1024
_VAL_BLOCKS = B // _BR
_GRID = (K + _BR - 1) // _BR


def _push_body(bank_ref, value_ref, out_ref):
    i = pl.program_id(0)

    @pl.when(i < _VAL_BLOCKS)
    def _():
        out_ref[...] = value_ref[...]

    @pl.when(i >= _VAL_BLOCKS)
    def _():
        out_ref[...] = bank_ref[...]


@jax.jit
def kernel(bank, value):
    return pl.pallas_call(
        _push_body,
        grid=(_GRID,),
        in_specs=[
            pl.BlockSpec((_BR, DIM), lambda i: (jnp.maximum(i, _VAL_BLOCKS), 0)),
            pl.BlockSpec((_BR, DIM), lambda i: (jnp.minimum(i, _VAL_BLOCKS - 1), 0)),
        ],
        out_specs=pl.BlockSpec((_BR, DIM), lambda i: (i, 0)),
        out_shape=jax.ShapeDtypeStruct((K, DIM), jnp.float32),
    )(bank, value)


# speedup vs baseline: 31.0788x; 1.4275x over previous
import jax
import jax.numpy as jnp
from jax.experimental import pallas as pl
from jax.experimental.pallas import tpu as pltpu

K = 100000
DIM = 128
B = 16384

_BR = 2048
_VAL_BLOCKS = B // _BR
_GRID = (K + _BR - 1) // _BR


def _push_body(bank_ref, value_ref, out_ref):
    i = pl.program_id(0)

    @pl.when(i < _VAL_BLOCKS)
    def _():
        out_ref[...] = value_ref[...]

    @pl.when(i >= _VAL_BLOCKS)
    def _():
        out_ref[...] = bank_ref[...]


@jax.jit
def kernel(bank, value):
    return pl.pallas_call(
        _push_body,
        grid=(_GRID,),
        in_specs=[
            pl.BlockSpec((_BR, DIM), lambda i: (jnp.maximum(i, _VAL_BLOCKS), 0)),
            pl.BlockSpec((_BR, DIM), lambda i: (jnp.minimum(i, _VAL_BLOCKS - 1), 0)),
        ],
        out_specs=pl.BlockSpec((_BR, DIM), lambda i: (i, 0)),
        out_shape=jax.ShapeDtypeStruct((K, DIM), jnp.float32),
    )(bank, value)


# speedup vs baseline: 42.7557x; 1.3757x over previous
import jax
import jax.numpy as jnp
from jax.experimental import pallas as pl
from jax.experimental.pallas import tpu as pltpu

K = 100000
DIM = 128
B = 16384

_BR = 4096
_VAL_BLOCKS = B // _BR
_GRID = (K + _BR - 1) // _BR


def _push_body(bank_ref, value_ref, out_ref):
    i = pl.program_id(0)

    @pl.when(i < _VAL_BLOCKS)
    def _():
        out_ref[...] = value_ref[...]

    @pl.when(i >= _VAL_BLOCKS)
    def _():
        out_ref[...] = bank_ref[...]


@jax.jit
def kernel(bank, value):
    return pl.pallas_call(
        _push_body,
        grid=(_GRID,),
        in_specs=[
            pl.BlockSpec((_BR, DIM), lambda i: (jnp.maximum(i, _VAL_BLOCKS), 0)),
            pl.BlockSpec((_BR, DIM), lambda i: (jnp.minimum(i, _VAL_BLOCKS - 1), 0)),
        ],
        out_specs=pl.BlockSpec((_BR, DIM), lambda i: (i, 0)),
        out_shape=jax.ShapeDtypeStruct((K, DIM), jnp.float32),
    )(bank, value)


# speedup vs baseline: 47.5887x; 1.1130x over previous
import jax
import jax.numpy as jnp
from jax.experimental import pallas as pl
from jax.experimental.pallas import tpu as pltpu

K = 100000
DIM = 128
B = 16384

_BR = 8192
_VAL_BLOCKS = B // _BR
_GRID = (K + _BR - 1) // _BR


def _push_body(bank_ref, value_ref, out_ref):
    i = pl.program_id(0)

    @pl.when(i < _VAL_BLOCKS)
    def _():
        out_ref[...] = value_ref[...]

    @pl.when(i >= _VAL_BLOCKS)
    def _():
        out_ref[...] = bank_ref[...]


@jax.jit
def kernel(bank, value):
    return pl.pallas_call(
        _push_body,
        grid=(_GRID,),
        in_specs=[
            pl.BlockSpec((_BR, DIM), lambda i: (jnp.maximum(i, _VAL_BLOCKS), 0)),
            pl.BlockSpec((_BR, DIM), lambda i: (jnp.minimum(i, _VAL_BLOCKS - 1), 0)),
        ],
        out_specs=pl.BlockSpec((_BR, DIM), lambda i: (i, 0)),
        out_shape=jax.ShapeDtypeStruct((K, DIM), jnp.float32),
    )(bank, value)
